# baseline (speedup 1.0000x reference)
import functools

import jax
import jax.numpy as jnp
from jax import lax
from jax.experimental import pallas as pl
from jax.experimental.pallas import tpu as pltpu
from jax.experimental.pallas import tpu_sc as plsc

E = 1_600_000
N = 50_000
D_EDGE = 16
D_NODE = 128
D_IN = 272
D_OUT = 128

HALF = E * D_EDGE // 2
J_TOTAL = E // 128
J_TC_SPLIT = 12_100
J_PER_W = (J_TOTAL - J_TC_SPLIT) // 16
FLAT_PER_W = J_PER_W * 1024
JB = 5
CHUNK = JB * 1024
NCHUNKS = FLAT_PER_W // CHUNK

_SC_MESH = plsc.VectorSubcoreMesh(core_axis_name="c", subcore_axis_name="s")


@functools.partial(
    pl.kernel,
    out_type=jax.ShapeDtypeStruct((32, 128), jnp.float32),
    mesh=_SC_MESH,
    scratch_types=[
        pltpu.VMEM((CHUNK,), jnp.float32),
        pltpu.VMEM((CHUNK,), jnp.float32),
        pltpu.VMEM((128,), jnp.float32),
        pltpu.SemaphoreType.DMA,
        pltpu.SemaphoreType.DMA,
    ],
)
def _sc_edge(edge_hbm, out_hbm, eb0, eb1, accbuf, sem0, sem1):
    wid = lax.axis_index("s") * 2 + lax.axis_index("c")
    t = wid // 16
    w16 = wid % 16
    base = t * HALF + J_TC_SPLIT * 1024 + w16 * FLAT_PER_W

    bufs = ((eb0, sem0), (eb1, sem1))
    for c in range(min(2, NCHUNKS)):
        eb, sem = bufs[c % 2]
        pltpu.async_copy(edge_hbm.at[pl.ds(base + c * CHUNK, CHUNK)], eb, sem)

    acc = [jnp.zeros((16,), jnp.float32)] * 8
    for c in range(NCHUNKS):
        eb, sem = bufs[c % 2]
        pltpu.make_async_copy(edge_hbm.at[pl.ds(base, CHUNK)], eb, sem).wait()
        for jj in range(JB):
            for k in range(8):
                a = acc[k]
                for q in range(8):
                    a = a + eb[pl.ds(jj * 1024 + k * 128 + q * 16, 16)]
                acc[k] = a
        if c + 2 < NCHUNKS:
            pltpu.async_copy(
                edge_hbm.at[pl.ds(base + (c + 2) * CHUNK, CHUNK)], eb, sem
            )
    for k in range(8):
        accbuf[pl.ds(k * 16, 16)] = acc[k]
    pltpu.sync_copy(accbuf, out_hbm.at[wid])


G = 50
BN = N // G
EW = J_TC_SPLIT * 128 // G
AN = 40
AE = 1408


def _tc_body(node_ref, edge_ref, part_ref, glob_ref, w_ref, b_ref,
             out_ref, acc_n, acc_e):
    i = pl.program_id(0)

    @pl.when(i == 0)
    def _init():
        acc_n[...] = jnp.zeros_like(acc_n)
        acc_e[...] = jnp.zeros_like(acc_e)

    eb = edge_ref[...]
    ae = acc_e[...]
    for k in range(EW // AE):
        ae = ae + eb[:, k * AE:(k + 1) * AE]
    acc_e[...] = ae

    nb = node_ref[...]
    an = acc_n[...]
    for k in range(BN // AN):
        an = an + nb[k * AN:(k + 1) * AN, :]
    acc_n[...] = an

    @pl.when(i == G - 1)
    def _final():
        n128 = jnp.sum(acc_n[...], axis=0, keepdims=True)
        te2 = jnp.sum(acc_e[...].reshape(D_EDGE, AE // 128, 128), axis=1)
        ones_r = jnp.ones((1, 128), jnp.float32)
        te_row = jax.lax.dot_general(
            ones_r, te2, (((1,), (1,)), ((), ())),
            preferred_element_type=jnp.float32,
        )
        p = part_ref[...]
        s0 = jnp.sum(p[:16, :], axis=0, keepdims=True)
        s1 = jnp.sum(p[16:, :], axis=0, keepdims=True)
        i_idx = jax.lax.broadcasted_iota(jnp.int32, (128, D_EDGE), 0)
        f_idx = jax.lax.broadcasted_iota(jnp.int32, (128, D_EDGE), 1)
        fold0 = (i_idx // 16 == f_idx).astype(jnp.float32)
        fold1 = (i_idx // 16 == f_idx - 8).astype(jnp.float32)
        mm = (((1,), (0,)), ((), ()))
        e16 = (
            jax.lax.dot_general(s0, fold0, mm, preferred_element_type=jnp.float32)
            + jax.lax.dot_general(s1, fold1, mm, preferred_element_type=jnp.float32)
            + te_row
        )
        x = jnp.concatenate(
            [e16 * (1.0 / E),
             n128 * (1.0 / N),
             glob_ref[...]],
            axis=1,
        )
        out_ref[...] = jax.lax.dot_general(
            x, w_ref[...], (((1,), (0,)), ((), ())),
            preferred_element_type=jnp.float32,
        ) + b_ref[...]


def kernel(edge_attrs, node_attrs, global_attr, W, b):
    edge_4d = edge_attrs.T.reshape(2, 8, J_TOTAL, 128).transpose(0, 2, 1, 3)
    edge_flat = edge_4d.reshape(-1)
    partials = _sc_edge(edge_flat)

    edge_t = edge_attrs.T
    glob2 = global_attr.reshape(1, D_NODE)
    b2 = b.reshape(1, D_OUT)
    out = pl.pallas_call(
        _tc_body,
        grid=(G,),
        in_specs=[
            pl.BlockSpec((BN, D_NODE), lambda i: (i, 0)),
            pl.BlockSpec((D_EDGE, EW), lambda i: (0, i)),
            pl.BlockSpec((32, 128), lambda i: (0, 0)),
            pl.BlockSpec((1, D_NODE), lambda i: (0, 0)),
            pl.BlockSpec((D_IN, D_OUT), lambda i: (0, 0)),
            pl.BlockSpec((1, D_OUT), lambda i: (0, 0)),
        ],
        out_specs=pl.BlockSpec((1, D_OUT), lambda i: (0, 0)),
        out_shape=jax.ShapeDtypeStruct((1, D_OUT), jnp.float32),
        scratch_shapes=[
            pltpu.VMEM((AN, D_NODE), jnp.float32),
            pltpu.VMEM((D_EDGE, AE), jnp.float32),
        ],
    )(node_attrs, edge_t, partials, glob2, W, b2)
    return out.reshape(D_OUT)


# speedup vs baseline: 1.2145x; 1.2145x over previous
import functools

import jax
import jax.numpy as jnp
from jax import lax
from jax.experimental import pallas as pl
from jax.experimental.pallas import tpu as pltpu
from jax.experimental.pallas import tpu_sc as plsc

E = 1_600_000
N = 50_000
D_EDGE = 16
D_NODE = 128
D_IN = 272
D_OUT = 128

HALF = E * D_EDGE // 2
J_TOTAL = E // 128
J_TC_SPLIT = 12_100
J_PER_W = (J_TOTAL - J_TC_SPLIT) // 16
FLAT_PER_W = J_PER_W * 1024
JB = 5
CHUNK = JB * 1024
NCHUNKS = FLAT_PER_W // CHUNK

_SC_MESH = plsc.VectorSubcoreMesh(core_axis_name="c", subcore_axis_name="s")


@functools.partial(
    pl.kernel,
    out_type=jax.ShapeDtypeStruct((32, 128), jnp.float32),
    mesh=_SC_MESH,
    scratch_types=[
        pltpu.VMEM((CHUNK,), jnp.float32),
        pltpu.VMEM((CHUNK,), jnp.float32),
        pltpu.VMEM((128,), jnp.float32),
        pltpu.SemaphoreType.DMA,
        pltpu.SemaphoreType.DMA,
    ],
)
def _sc_edge(edge_hbm, out_hbm, eb0, eb1, accbuf, sem0, sem1):
    wid = lax.axis_index("s") * 2 + lax.axis_index("c")
    t = wid // 16
    w16 = wid % 16
    base = t * HALF + J_TC_SPLIT * 1024 + w16 * FLAT_PER_W

    bufs = ((eb0, sem0), (eb1, sem1))
    for c in range(min(2, NCHUNKS)):
        eb, sem = bufs[c % 2]
        pltpu.async_copy(edge_hbm.at[pl.ds(base + c * CHUNK, CHUNK)], eb, sem)

    acc = [jnp.zeros((16,), jnp.float32)] * 8
    for c in range(NCHUNKS):
        eb, sem = bufs[c % 2]
        pltpu.make_async_copy(edge_hbm.at[pl.ds(base, CHUNK)], eb, sem).wait()
        for jj in range(JB):
            for k in range(8):
                a = acc[k]
                for q in range(8):
                    a = a + eb[pl.ds(jj * 1024 + k * 128 + q * 16, 16)]
                acc[k] = a
        if c + 2 < NCHUNKS:
            pltpu.async_copy(
                edge_hbm.at[pl.ds(base + (c + 2) * CHUNK, CHUNK)], eb, sem
            )
    for k in range(8):
        accbuf[pl.ds(k * 16, 16)] = acc[k]
    pltpu.sync_copy(accbuf, out_hbm.at[wid])


G = 50
BN = N // G
EW = J_TC_SPLIT * 128 // G
AN = 40
AE = 1408


def _tc_body(node_ref, edge_ref, out_ref, acc_n, acc_e):
    i = pl.program_id(0)

    @pl.when(i == 0)
    def _init():
        acc_n[...] = jnp.zeros_like(acc_n)
        acc_e[...] = jnp.zeros_like(acc_e)

    eb = edge_ref[...]
    ae = acc_e[...]
    for k in range(EW // AE):
        ae = ae + eb[:, k * AE:(k + 1) * AE]
    acc_e[...] = ae

    nb = node_ref[...]
    an = acc_n[...]
    for k in range(BN // AN):
        an = an + nb[k * AN:(k + 1) * AN, :]
    acc_n[...] = an

    @pl.when(i == G - 1)
    def _final():
        n128 = jnp.sum(acc_n[...], axis=0, keepdims=True)
        te2 = jnp.sum(acc_e[...].reshape(D_EDGE, AE // 128, 128), axis=1)
        ones_r = jnp.ones((1, 128), jnp.float32)
        te_row = jax.lax.dot_general(
            ones_r, te2, (((1,), (1,)), ((), ())),
            preferred_element_type=jnp.float32,
        )
        out_ref[0:1, :D_EDGE] = te_row
        out_ref[1:2, :] = n128


def _tc_combine_body(part_ref, main_ref, glob_ref, w_ref, b_ref, out_ref):
    p = part_ref[...]
    s0 = jnp.sum(p[:16, :], axis=0, keepdims=True)
    s1 = jnp.sum(p[16:, :], axis=0, keepdims=True)
    i_idx = jax.lax.broadcasted_iota(jnp.int32, (128, D_EDGE), 0)
    f_idx = jax.lax.broadcasted_iota(jnp.int32, (128, D_EDGE), 1)
    fold0 = (i_idx // 16 == f_idx).astype(jnp.float32)
    fold1 = (i_idx // 16 == f_idx - 8).astype(jnp.float32)
    mm = (((1,), (0,)), ((), ()))
    e16 = (
        jax.lax.dot_general(s0, fold0, mm, preferred_element_type=jnp.float32)
        + jax.lax.dot_general(s1, fold1, mm, preferred_element_type=jnp.float32)
        + main_ref[0:1, :D_EDGE]
    )
    x = jnp.concatenate(
        [e16 * (1.0 / E),
         main_ref[1:2, :] * (1.0 / N),
         glob_ref[...]],
        axis=1,
    )
    out_ref[...] = jax.lax.dot_general(
        x, w_ref[...], (((1,), (0,)), ((), ())),
        preferred_element_type=jnp.float32,
    ) + b_ref[...]


def kernel(edge_attrs, node_attrs, global_attr, W, b):
    edge_4d = edge_attrs.T.reshape(2, 8, J_TOTAL, 128).transpose(0, 2, 1, 3)
    edge_flat = edge_4d.reshape(-1)
    partials = _sc_edge(edge_flat)

    edge_t = edge_attrs.T
    main = pl.pallas_call(
        _tc_body,
        grid=(G,),
        in_specs=[
            pl.BlockSpec((BN, D_NODE), lambda i: (i, 0)),
            pl.BlockSpec((D_EDGE, EW), lambda i: (0, i)),
        ],
        out_specs=pl.BlockSpec((2, D_NODE), lambda i: (0, 0)),
        out_shape=jax.ShapeDtypeStruct((2, D_NODE), jnp.float32),
        scratch_shapes=[
            pltpu.VMEM((AN, D_NODE), jnp.float32),
            pltpu.VMEM((D_EDGE, AE), jnp.float32),
        ],
    )(node_attrs, edge_t)

    glob2 = global_attr.reshape(1, D_NODE)
    b2 = b.reshape(1, D_OUT)
    out = pl.pallas_call(
        _tc_combine_body,
        out_shape=jax.ShapeDtypeStruct((1, D_OUT), jnp.float32),
    )(partials, main, glob2, W, b2)
    return out.reshape(D_OUT)
